# baseline (speedup 1.0000x reference)
import functools

import jax
import jax.numpy as jnp
from jax import lax
from jax.experimental import pallas as pl
from jax.experimental.pallas import tpu as pltpu
from jax.experimental.pallas import tpu_sc as plsc

_L = 16
_W = 128


@functools.partial(jax.jit, static_argnums=(3, 4))
def _gather_loss(probT, target, reward, n, c):
    ns = 16
    rows_per = n // ns
    n_win = rows_per // _W
    n_buf = 4

    mesh = plsc.VectorSubcoreMesh(core_axis_name="c", subcore_axis_name="s")

    @functools.partial(
        pl.kernel,
        mesh=mesh,
        out_type=jax.ShapeDtypeStruct((_L,), jnp.float32),
        compiler_params=pltpu.CompilerParams(needs_layout_passes=False),
        scratch_types=[
            pltpu.VMEM((rows_per,), jnp.int32),
            pltpu.VMEM((rows_per,), jnp.float32),
            pltpu.VMEM((n_buf, _W, _W), jnp.float32),
            pltpu.VMEM((_L,), jnp.float32),
            pltpu.VMEM((1, _L), jnp.float32),
            pltpu.VMEM((_L,), jnp.int32),
            pltpu.VMEM_SHARED((1, _L), jnp.float32),
            pltpu.SemaphoreType.DMA,
        ],
    )
    def body(probT_hbm, tgt_hbm, rew_hbm, out_hbm,
             tgt_v, rew_v, win_v, acc_v, acc1_v, zidx_v, shared, sem):
        cid = lax.axis_index("c")
        sid = lax.axis_index("s")
        base = sid * rows_per

        pltpu.sync_copy(tgt_hbm.at[pl.ds(base, rows_per)], tgt_v)
        pltpu.sync_copy(rew_hbm.at[pl.ds(base, rows_per)], rew_v)

        lane = lax.iota(jnp.int32, _L)
        zidx_v[...] = lane * 0

        @pl.when(sid == 0)
        def _():
            acc_v[...] = jnp.zeros((_L,), jnp.float32)
            pltpu.sync_copy(acc_v, shared.at[0])

        def start(w, j):
            return pltpu.async_copy(
                probT_hbm.at[tgt_v.at[pl.ds(w * _W, _W)],
                             pl.ds(base + w * _W, _W)],
                win_v.at[j],
                sem)

        acc = jnp.zeros((_L,), jnp.float32)
        for b in range(n_win // n_buf):
            copies = [start(b * n_buf + j, j) for j in range(n_buf)]
            for cp in copies:
                cp.wait()
            for j in range(n_buf):
                w = b * n_buf + j
                for k in range(_W // _L):
                    diag = k * _L + lane
                    d = plsc.load_gather(win_v.at[j], [diag, diag])
                    acc = acc + d * rew_v[pl.ds(w * _W + k * _L, _L)]
        acc1_v[0] = acc

        plsc.subcore_barrier()
        pltpu.sync_copy(acc1_v, shared.at[zidx_v.at[pl.ds(0, 1)]], add=True)
        plsc.subcore_barrier()

        @pl.when(jnp.logical_and(sid == 0, cid == 0))
        def _():
            pltpu.sync_copy(shared.at[0], acc_v)
            for shift in (8, 4, 2, 1):
                x = acc_v[...]
                perm = plsc.load_gather(acc_v, [(lane + shift) & (_L - 1)])
                acc_v[...] = x + perm
            acc_v[...] = acc_v[...] * (-1.0 / n)
            pltpu.sync_copy(acc_v, out_hbm)

    return body(probT, target, reward)


def kernel(prob, target, reward):
    n, c = prob.shape
    out = _gather_loss(prob.T, target.astype(jnp.int32), reward, n, c)
    return out[0]


# speedup vs baseline: 1.0873x; 1.0873x over previous
import functools

import jax
import jax.numpy as jnp
from jax import lax
from jax.experimental import pallas as pl
from jax.experimental.pallas import tpu as pltpu
from jax.experimental.pallas import tpu_sc as plsc

_L = 16
_W = 128


@functools.partial(jax.jit, static_argnums=(3, 4))
def _gather_loss(probT, target, reward, n, c):
    ns = 16
    nw = 2 * ns
    rows_per = n // nw
    n_win = rows_per // _W
    n_buf = 4

    mesh = plsc.VectorSubcoreMesh(core_axis_name="c", subcore_axis_name="s")

    @functools.partial(
        pl.kernel,
        mesh=mesh,
        out_type=jax.ShapeDtypeStruct((2, _L), jnp.float32),
        compiler_params=pltpu.CompilerParams(needs_layout_passes=False),
        scratch_types=[
            pltpu.VMEM((rows_per,), jnp.int32),
            pltpu.VMEM((rows_per,), jnp.float32),
            pltpu.VMEM((n_buf, _W, _W), jnp.float32),
            pltpu.VMEM((_L,), jnp.float32),
            pltpu.VMEM((1, _L), jnp.float32),
            pltpu.VMEM((_L,), jnp.int32),
            pltpu.VMEM_SHARED((1, _L), jnp.float32),
            pltpu.SemaphoreType.DMA,
        ],
    )
    def body(probT_hbm, tgt_hbm, rew_hbm, out_hbm,
             tgt_v, rew_v, win_v, acc_v, acc1_v, zidx_v, shared, sem):
        cid = lax.axis_index("c")
        sid = lax.axis_index("s")
        base = (sid * 2 + cid) * rows_per

        pltpu.sync_copy(tgt_hbm.at[pl.ds(base, rows_per)], tgt_v)
        pltpu.sync_copy(rew_hbm.at[pl.ds(base, rows_per)], rew_v)

        lane = lax.iota(jnp.int32, _L)
        zidx_v[...] = lane * 0

        @pl.when(sid == 0)
        def _():
            acc_v[...] = jnp.zeros((_L,), jnp.float32)
            pltpu.sync_copy(acc_v, shared.at[0])

        def start(w, j):
            return pltpu.async_copy(
                probT_hbm.at[tgt_v.at[pl.ds(w * _W, _W)],
                             pl.ds(base + w * _W, _W)],
                win_v.at[j],
                sem)

        acc = jnp.zeros((_L,), jnp.float32)
        for b in range(n_win // n_buf):
            copies = [start(b * n_buf + j, j) for j in range(n_buf)]
            for cp in copies:
                cp.wait()
            for j in range(n_buf):
                w = b * n_buf + j
                for k in range(_W // _L):
                    diag = k * _L + lane
                    d = plsc.load_gather(win_v.at[j], [diag, diag])
                    acc = acc + d * rew_v[pl.ds(w * _W + k * _L, _L)]
        acc1_v[0] = acc

        plsc.subcore_barrier()
        pltpu.sync_copy(acc1_v, shared.at[zidx_v.at[pl.ds(0, 1)]], add=True)
        plsc.subcore_barrier()

        @pl.when(sid == 0)
        def _():
            pltpu.sync_copy(shared.at[0], acc_v)
            for shift in (8, 4, 2, 1):
                x = acc_v[...]
                perm = plsc.load_gather(acc_v, [(lane + shift) & (_L - 1)])
                acc_v[...] = x + perm
            acc_v[...] = acc_v[...] * (-1.0 / n)
            pltpu.sync_copy(acc_v, out_hbm.at[cid])

    return body(probT, target, reward)


def kernel(prob, target, reward):
    n, c = prob.shape
    out = _gather_loss(prob.T, target.astype(jnp.int32), reward, n, c)
    return out[0, 0] + out[1, 0]


# speedup vs baseline: 1.1569x; 1.0641x over previous
import functools

import jax
import jax.numpy as jnp
from jax import lax
from jax.experimental import pallas as pl
from jax.experimental.pallas import tpu as pltpu
from jax.experimental.pallas import tpu_sc as plsc

_L = 16
_W = 128


@functools.partial(jax.jit, static_argnums=(3, 4))
def _gather_loss(probT, target, reward, n, c):
    ns = 16
    nw = 2 * ns
    rows_per = n // nw
    n_win = rows_per // _W
    n_buf = 4

    mesh = plsc.VectorSubcoreMesh(core_axis_name="c", subcore_axis_name="s")

    @functools.partial(
        pl.kernel,
        mesh=mesh,
        out_type=jax.ShapeDtypeStruct((2, _L), jnp.float32),
        compiler_params=pltpu.CompilerParams(needs_layout_passes=False,
                                             skip_device_barrier=True),
        scratch_types=[
            pltpu.VMEM((rows_per,), jnp.int32),
            pltpu.VMEM((rows_per,), jnp.float32),
            pltpu.VMEM((n_buf, _W, _W), jnp.float32),
            pltpu.VMEM((_L,), jnp.float32),
            pltpu.VMEM((1, _L), jnp.float32),
            pltpu.VMEM((_L,), jnp.int32),
            pltpu.VMEM_SHARED((1, _L), jnp.float32),
            pltpu.SemaphoreType.DMA,
        ],
    )
    def body(probT_hbm, tgt_hbm, rew_hbm, out_hbm,
             tgt_v, rew_v, win_v, acc_v, acc1_v, zidx_v, shared, sem):
        cid = lax.axis_index("c")
        sid = lax.axis_index("s")
        base = (sid * 2 + cid) * rows_per

        pltpu.sync_copy(tgt_hbm.at[pl.ds(base, rows_per)], tgt_v)
        rew_cp = pltpu.async_copy(rew_hbm.at[pl.ds(base, rows_per)], rew_v,
                                  sem)

        lane = lax.iota(jnp.int32, _L)
        zidx_v[...] = lane * 0

        @pl.when(sid == 0)
        def _():
            acc_v[...] = jnp.zeros((_L,), jnp.float32)
            pltpu.sync_copy(acc_v, shared.at[0])

        def start(w, j):
            return pltpu.async_copy(
                probT_hbm.at[tgt_v.at[pl.ds(w * _W, _W)],
                             pl.ds(base + w * _W, _W)],
                win_v.at[j],
                sem)

        acc = jnp.zeros((_L,), jnp.float32)
        for b in range(n_win // n_buf):
            copies = [start(b * n_buf + j, j) for j in range(n_buf)]
            rew_cp = rew_cp.wait() if b == 0 else None
            for cp in copies:
                cp.wait()
            for j in range(n_buf):
                w = b * n_buf + j

                def ext(k, a, _j=j, _w=w):
                    diag = k * _L + lane
                    d = plsc.load_gather(win_v.at[_j], [diag, diag])
                    return a + d * rew_v[pl.ds(_w * _W + k * _L, _L)]

                acc = lax.fori_loop(0, _W // _L, ext, acc)
        acc1_v[0] = acc

        plsc.subcore_barrier()
        pltpu.sync_copy(acc1_v, shared.at[zidx_v.at[pl.ds(0, 1)]], add=True)
        plsc.subcore_barrier()

        @pl.when(sid == 0)
        def _():
            pltpu.sync_copy(shared.at[0], acc_v)
            for shift in (8, 4, 2, 1):
                x = acc_v[...]
                perm = plsc.load_gather(acc_v, [(lane + shift) & (_L - 1)])
                acc_v[...] = x + perm
            acc_v[...] = acc_v[...] * (-1.0 / n)
            pltpu.sync_copy(acc_v, out_hbm.at[cid])

    return body(probT, target, reward)


def kernel(prob, target, reward):
    n, c = prob.shape
    out = _gather_loss(prob.T, target.astype(jnp.int32), reward, n, c)
    return out[0, 0] + out[1, 0]
